# baseline (speedup 1.0000x reference)
import functools

import jax
import jax.numpy as jnp
from jax import lax
from jax.experimental import pallas as pl
from jax.experimental.pallas import tpu as pltpu
from jax.experimental.pallas import tpu_sc as plsc

_N = 10000
_NP = 10048
_E = 320000
_EP = 327680
_D = 128
_K = 128
_TCHUNK = _EP // _K
_B = 3
_CA = 128
_CB = 32
_RPT = 632
_RPT_LAST = _NP - 15 * _RPT


@functools.cache
def _sc_agg_kernel():
    mesh = plsc.VectorSubcoreMesh(core_axis_name="c", subcore_axis_name="s")

    @functools.partial(
        pl.kernel,
        mesh=mesh,
        out_type=jax.ShapeDtypeStruct((2, _NP, _D), jnp.float32),
        scratch_types=[
            pltpu.VMEM((_B, 2, _K), jnp.int32),
            pltpu.VMEM((_K, _D), jnp.float32),
            pltpu.VMEM((_K, _D), jnp.float32),
            pltpu.VMEM((_K, _D), jnp.float32),
            pltpu.VMEM_SHARED((_NP, _D), jnp.float32),
            pltpu.SemaphoreType.DMA,
            pltpu.SemaphoreType.DMA,
            pltpu.SemaphoreType.DMA,
            pltpu.SemaphoreType.DMA,
            pltpu.SemaphoreType.DMA,
            pltpu.SemaphoreType.DMA,
        ],
    )
    def _sc_agg(h_hbm, idx_hbm, out_hbm,
                idx_v, rows0, rows1, rows2, agg_sh,
                rs0, rs1, rs2, is0, is1, is2):
        c = lax.axis_index("c")
        s = lax.axis_index("s")
        bufs = (rows0, rows1, rows2)
        rsems = (rs0, rs1, rs2)
        isems = (is0, is1, is2)
        start = jnp.where(c == 0, s * _CA, 16 * _CA + s * _CB)
        n = jnp.where(c == 0, _CA, _CB)
        zv = jnp.zeros((16,), jnp.float32)

        def zbody(i, carry):
            for jj in range(8):
                rows0[i, pl.ds(jj * 16, 16)] = zv
            return carry

        lax.fori_loop(0, _K, zbody, 0, unroll=False)
        zbase = s * _RPT
        for r in range(4):
            pltpu.sync_copy(rows0, agg_sh.at[pl.ds(zbase + r * _K, _K)])

        @pl.when(s < 15)
        def _ztail():
            pltpu.sync_copy(rows0.at[pl.ds(0, _RPT - 4 * _K)],
                            agg_sh.at[pl.ds(zbase + 4 * _K, _RPT - 4 * _K)])

        @pl.when(s == 15)
        def _ztail_last():
            pltpu.sync_copy(rows0.at[pl.ds(0, _RPT_LAST - 4 * _K)],
                            agg_sh.at[pl.ds(zbase + 4 * _K,
                                            _RPT_LAST - 4 * _K)])

        plsc.subcore_barrier()

        for m in range(_B):
            pltpu.async_copy(idx_hbm.at[start + m], idx_v.at[m], isems[m])
        for m in range(_B - 1):
            pltpu.make_async_copy(idx_hbm.at[start + m], idx_v.at[m],
                                  isems[m]).wait()
            pltpu.async_copy(h_hbm.at[idx_v.at[m, 0]], bufs[m], rsems[m])

        def body(j0, carry):
            for b in range(_B):
                j = j0 + b
                bb = (b + _B - 1) % _B
                pltpu.make_async_copy(h_hbm.at[idx_v.at[b, 0]], bufs[b],
                                      rsems[b]).wait()
                pltpu.sync_copy(bufs[b], agg_sh.at[idx_v.at[b, 1]], add=True)

                @pl.when(j + _B < n)
                def _pf():
                    pltpu.async_copy(idx_hbm.at[start + j + _B], idx_v.at[b],
                                     isems[b])

                @pl.when(j + _B - 1 < n)
                def _fire():
                    pltpu.make_async_copy(idx_hbm.at[start + j + _B - 1],
                                          idx_v.at[bb], isems[bb]).wait()
                    pltpu.async_copy(h_hbm.at[idx_v.at[bb, 0]], bufs[bb],
                                     rsems[bb])
            return carry

        lax.fori_loop(0, n // _B, lambda i, car: body(i * _B, car), 0,
                      unroll=False)
        rem = n % _B
        for m in range(_B - 1):
            @pl.when(m < rem)
            def _tail():
                pltpu.make_async_copy(h_hbm.at[idx_v.at[m, 0]], bufs[m],
                                      rsems[m]).wait()
                pltpu.sync_copy(bufs[m], agg_sh.at[idx_v.at[m, 1]], add=True)
        plsc.subcore_barrier()

        @pl.when(s < 15)
        def _drain():
            pltpu.sync_copy(agg_sh.at[pl.ds(s * _RPT, _RPT)],
                            out_hbm.at[c, pl.ds(s * _RPT, _RPT)])

        @pl.when(s == 15)
        def _drain_last():
            pltpu.sync_copy(agg_sh.at[pl.ds(15 * _RPT, _RPT_LAST)],
                            out_hbm.at[c, pl.ds(15 * _RPT, _RPT_LAST)])

    return _sc_agg


def _lin_body(p_ref, W_ref, b_ref, o_ref):
    acc = p_ref[0] + p_ref[1]
    o_ref[...] = jnp.maximum(
        jnp.dot(acc, W_ref[...], preferred_element_type=jnp.float32)
        + b_ref[...], 0.0)


def _linear_relu(p, W, b2d):
    blk = 1256
    return pl.pallas_call(
        _lin_body,
        grid=(_NP // blk,),
        in_specs=[
            pl.BlockSpec((2, blk, _D), lambda i: (0, i, 0)),
            pl.BlockSpec((_D, _D), lambda i: (0, 0)),
            pl.BlockSpec((1, _D), lambda i: (0, 0)),
        ],
        out_specs=pl.BlockSpec((blk, _D), lambda i: (i, 0)),
        out_shape=jax.ShapeDtypeStruct((_NP, _D), jnp.float32),
    )(p, W, b2d)


def _heads_body(h_ref, Wpg_ref, bpg_ref, Wpd_ref, bpd_ref, Wv_ref, bv_ref,
                pi_ref, pid_ref, v_ref):
    h = h_ref[...]
    rid = lax.broadcasted_iota(jnp.int32, (_NP, _D), 0)
    hm = jnp.where(rid < _N, h, 0.0)
    mN = jnp.sum(hm, axis=0, keepdims=True) * (1.0 / _N)
    pi_ref[...] = (jnp.dot(h, Wpg_ref[...], preferred_element_type=jnp.float32)
                   + bpg_ref[...])
    pid_ref[...] = (jnp.dot(mN, Wpd_ref[...], preferred_element_type=jnp.float32)
                    + bpd_ref[...])
    v_ref[...] = (jnp.dot(mN, Wv_ref[...], preferred_element_type=jnp.float32)
                  + bv_ref[...])


def _heads(h, Wpg, bpg, Wpd, bpd, Wv, bv):
    return pl.pallas_call(
        _heads_body,
        out_shape=[
            jax.ShapeDtypeStruct((_NP, 1), jnp.float32),
            jax.ShapeDtypeStruct((1, 1), jnp.float32),
            jax.ShapeDtypeStruct((1, 1), jnp.float32),
        ],
    )(h, Wpg, bpg.reshape(1, 1), Wpd, bpd.reshape(1, 1), Wv, bv.reshape(1, 1))


def kernel(x, edge_index, W1, b1, W2, b2, W3, b3, Wpg, bpg, Wpd, bpd, Wv, bv):
    src = edge_index[0]
    dst = edge_index[1]
    pad_e = _EP - _E
    src_p = jnp.concatenate(
        [src, jnp.zeros((pad_e,), jnp.int32)]).reshape(_TCHUNK, 1, _K)
    dst_p = jnp.concatenate(
        [dst, jnp.full((pad_e,), _N, jnp.int32)]).reshape(_TCHUNK, 1, _K)
    idx_p = jnp.concatenate([src_p, dst_p], axis=1)
    h = jnp.concatenate([x, jnp.zeros((_NP - _N, _D), jnp.float32)], axis=0)
    for W, b in ((W1, b1), (W2, b2), (W3, b3)):
        p = _sc_agg_kernel()(h, idx_p)
        h = _linear_relu(p, W, b.reshape(1, _D))
    pi_nodes, pi_done, v = _heads(h, Wpg, bpg, Wpd, bpd, Wv, bv)
    pi = jnp.concatenate([pi_nodes[:_N], pi_done], axis=0)
    return (pi, v)


# speedup vs baseline: 1.4037x; 1.4037x over previous
import functools

import jax
import jax.numpy as jnp
from jax import lax
from jax.experimental import pallas as pl
from jax.experimental.pallas import tpu as pltpu
from jax.experimental.pallas import tpu_sc as plsc

_N = 10000
_NP = 10048
_E = 320000
_EP = 327680
_D = 128
_K = 128
_TCHUNK = _EP // _K
_B = 3
_CA = 148
_CB = 12
_RPT = 632
_RPT_LAST = _NP - 15 * _RPT


@functools.cache
def _sc_agg_kernel():
    mesh = plsc.VectorSubcoreMesh(core_axis_name="c", subcore_axis_name="s")

    @functools.partial(
        pl.kernel,
        mesh=mesh,
        out_type=jax.ShapeDtypeStruct((2, _NP, _D), jnp.float32),
        scratch_types=[
            pltpu.VMEM((_B, 2, _K), jnp.int32),
            pltpu.VMEM((_K, _D), jnp.float32),
            pltpu.VMEM((_K, _D), jnp.float32),
            pltpu.VMEM((_K, _D), jnp.float32),
            pltpu.VMEM_SHARED((_NP, _D), jnp.float32),
            pltpu.SemaphoreType.DMA,
            pltpu.SemaphoreType.DMA,
            pltpu.SemaphoreType.DMA,
            pltpu.SemaphoreType.DMA,
            pltpu.SemaphoreType.DMA,
            pltpu.SemaphoreType.DMA,
        ],
    )
    def _sc_agg(h_hbm, idx_hbm, out_hbm,
                idx_v, rows0, rows1, rows2, agg_sh,
                rs0, rs1, rs2, is0, is1, is2):
        c = lax.axis_index("c")
        s = lax.axis_index("s")
        bufs = (rows0, rows1, rows2)
        rsems = (rs0, rs1, rs2)
        isems = (is0, is1, is2)
        start = jnp.where(c == 0, s * _CA, 16 * _CA + s * _CB)
        n = jnp.where(c == 0, _CA, _CB)
        zv = jnp.zeros((16,), jnp.float32)

        def zbody(i, carry):
            for jj in range(8):
                rows0[i, pl.ds(jj * 16, 16)] = zv
            return carry

        lax.fori_loop(0, _K, zbody, 0, unroll=False)
        zbase = s * _RPT
        for r in range(4):
            pltpu.sync_copy(rows0, agg_sh.at[pl.ds(zbase + r * _K, _K)])

        @pl.when(s < 15)
        def _ztail():
            pltpu.sync_copy(rows0.at[pl.ds(0, _RPT - 4 * _K)],
                            agg_sh.at[pl.ds(zbase + 4 * _K, _RPT - 4 * _K)])

        @pl.when(s == 15)
        def _ztail_last():
            pltpu.sync_copy(rows0.at[pl.ds(0, _RPT_LAST - 4 * _K)],
                            agg_sh.at[pl.ds(zbase + 4 * _K,
                                            _RPT_LAST - 4 * _K)])

        plsc.subcore_barrier()

        for m in range(_B):
            pltpu.async_copy(idx_hbm.at[start + m], idx_v.at[m], isems[m])
        for m in range(_B - 1):
            pltpu.make_async_copy(idx_hbm.at[start + m], idx_v.at[m],
                                  isems[m]).wait()
            pltpu.async_copy(h_hbm.at[idx_v.at[m, 0]], bufs[m], rsems[m])

        def body(j0, carry):
            for b in range(_B):
                j = j0 + b
                bb = (b + _B - 1) % _B
                pltpu.make_async_copy(h_hbm.at[idx_v.at[b, 0]], bufs[b],
                                      rsems[b]).wait()
                pltpu.sync_copy(bufs[b], agg_sh.at[idx_v.at[b, 1]], add=True)

                @pl.when(j + _B < n)
                def _pf():
                    pltpu.async_copy(idx_hbm.at[start + j + _B], idx_v.at[b],
                                     isems[b])

                @pl.when(j + _B - 1 < n)
                def _fire():
                    pltpu.make_async_copy(idx_hbm.at[start + j + _B - 1],
                                          idx_v.at[bb], isems[bb]).wait()
                    pltpu.async_copy(h_hbm.at[idx_v.at[bb, 0]], bufs[bb],
                                     rsems[bb])
            return carry

        lax.fori_loop(0, n // _B, lambda i, car: body(i * _B, car), 0,
                      unroll=False)
        rem = n % _B
        for m in range(_B - 1):
            @pl.when(m < rem)
            def _tail():
                pltpu.make_async_copy(h_hbm.at[idx_v.at[m, 0]], bufs[m],
                                      rsems[m]).wait()
                pltpu.sync_copy(bufs[m], agg_sh.at[idx_v.at[m, 1]], add=True)
        plsc.subcore_barrier()

        @pl.when(s < 15)
        def _drain():
            pltpu.sync_copy(agg_sh.at[pl.ds(s * _RPT, _RPT)],
                            out_hbm.at[c, pl.ds(s * _RPT, _RPT)])

        @pl.when(s == 15)
        def _drain_last():
            pltpu.sync_copy(agg_sh.at[pl.ds(15 * _RPT, _RPT_LAST)],
                            out_hbm.at[c, pl.ds(15 * _RPT, _RPT_LAST)])

    return _sc_agg


def _lin_body(p_ref, W_ref, b_ref, o_ref):
    acc = p_ref[0] + p_ref[1]
    o_ref[...] = jnp.maximum(
        jnp.dot(acc, W_ref[...], preferred_element_type=jnp.float32)
        + b_ref[...], 0.0)


def _linear_relu(p, W, b2d):
    blk = 1256
    return pl.pallas_call(
        _lin_body,
        grid=(_NP // blk,),
        in_specs=[
            pl.BlockSpec((2, blk, _D), lambda i: (0, i, 0)),
            pl.BlockSpec((_D, _D), lambda i: (0, 0)),
            pl.BlockSpec((1, _D), lambda i: (0, 0)),
        ],
        out_specs=pl.BlockSpec((blk, _D), lambda i: (i, 0)),
        out_shape=jax.ShapeDtypeStruct((_NP, _D), jnp.float32),
    )(p, W, b2d)


def _heads_body(h_ref, Wpg_ref, bpg_ref, Wpd_ref, bpd_ref, Wv_ref, bv_ref,
                pi_ref, pid_ref, v_ref):
    h = h_ref[...]
    rid = lax.broadcasted_iota(jnp.int32, (_NP, _D), 0)
    hm = jnp.where(rid < _N, h, 0.0)
    mN = jnp.sum(hm, axis=0, keepdims=True) * (1.0 / _N)
    pi_ref[...] = (jnp.dot(h, Wpg_ref[...], preferred_element_type=jnp.float32)
                   + bpg_ref[...])
    pid_ref[...] = (jnp.dot(mN, Wpd_ref[...], preferred_element_type=jnp.float32)
                    + bpd_ref[...])
    v_ref[...] = (jnp.dot(mN, Wv_ref[...], preferred_element_type=jnp.float32)
                  + bv_ref[...])


def _heads(h, Wpg, bpg, Wpd, bpd, Wv, bv):
    return pl.pallas_call(
        _heads_body,
        out_shape=[
            jax.ShapeDtypeStruct((_NP, 1), jnp.float32),
            jax.ShapeDtypeStruct((1, 1), jnp.float32),
            jax.ShapeDtypeStruct((1, 1), jnp.float32),
        ],
    )(h, Wpg, bpg.reshape(1, 1), Wpd, bpd.reshape(1, 1), Wv, bv.reshape(1, 1))


def kernel(x, edge_index, W1, b1, W2, b2, W3, b3, Wpg, bpg, Wpd, bpd, Wv, bv):
    src = edge_index[0]
    dst = edge_index[1]
    pad_e = _EP - _E
    src_p = jnp.concatenate(
        [src, jnp.zeros((pad_e,), jnp.int32)]).reshape(_TCHUNK, 1, _K)
    dst_p = jnp.concatenate(
        [dst, jnp.full((pad_e,), _N, jnp.int32)]).reshape(_TCHUNK, 1, _K)
    idx_p = jnp.concatenate([src_p, dst_p], axis=1)
    h = jnp.concatenate([x, jnp.zeros((_NP - _N, _D), jnp.float32)], axis=0)
    for W, b in ((W1, b1), (W2, b2), (W3, b3)):
        p = _sc_agg_kernel()(h, idx_p)
        h = _linear_relu(p, W, b.reshape(1, _D))
    pi_nodes, pi_done, v = _heads(h, Wpg, bpg, Wpd, bpd, Wv, bv)
    pi = jnp.concatenate([pi_nodes[:_N], pi_done], axis=0)
    return (pi, v)
